# Initial kernel scaffold; baseline (speedup 1.0000x reference)
import jax
import jax.numpy as jnp
from jax.experimental import pallas as pl


def kernel(cost):
    raise NotImplementedError("write your pallas kernel here")


# speedup vs baseline: 5.2713x; 5.2713x over previous
import functools

import jax
import jax.numpy as jnp
from jax import lax
from jax.experimental import pallas as pl
from jax.experimental.pallas import tpu as pltpu
from jax.experimental.pallas import tpu_sc as plsc

D = 192
B = 4
N = 65536
R_TOTAL = B * N
NC = 2
NS = 16
L = 16
NW = NC * NS
RPW = R_TOTAL // NW
G = 128
NCH = RPW // G
NGRP = G // L
I = 4
U = 4


def _make_kernel():
    mesh = plsc.VectorSubcoreMesh(core_axis_name="c", subcore_axis_name="s")

    @functools.partial(
        pl.kernel,
        out_type=jax.ShapeDtypeStruct((R_TOTAL,), jnp.float32),
        mesh=mesh,
        compiler_params=pltpu.CompilerParams(needs_layout_passes=False),
        scratch_types=[
            pltpu.VMEM((G * D,), jnp.float32),
            pltpu.VMEM((G * D,), jnp.float32),
            pltpu.VMEM((RPW,), jnp.float32),
            pltpu.SemaphoreType.DMA,
            pltpu.SemaphoreType.DMA,
        ],
    )
    def disp_kernel(cost_hbm, out_hbm, buf_a, buf_b, out_v, sem_a, sem_b):
        wid = lax.axis_index("s") * NC + lax.axis_index("c")
        row0 = wid * RPW
        iota = lax.iota(jnp.int32, L)

        def start(c, buf, sem):
            pltpu.async_copy(
                cost_hbm.at[pl.ds((row0 + c * G) * D, G * D)], buf, sem)

        def wait(buf, sem):
            pltpu.make_async_copy(
                cost_hbm.at[pl.ds(0, G * D)], buf, sem).wait()

        neg = jnp.full((L,), -jnp.inf, dtype=jnp.float32)
        zeros = jnp.zeros((L,), dtype=jnp.int32)

        def compute(buf, c):
            for g0 in range(0, NGRP, I):
                rows_d = [(iota + (g0 + i) * L) * D for i in range(I)]

                def dbody(dd, carry):
                    st = list(carry)
                    for u in range(U):
                        d = dd * U + u
                        col = zeros + d
                        for i in range(I):
                            m1, m2, i1, i2 = st[i]
                            v = plsc.load_gather(buf, [rows_d[i] + col])
                            gt1 = v > m1
                            gt2 = v > m2
                            n_i2 = jnp.where(gt1, i1, jnp.where(gt2, col, i2))
                            n_m2 = jnp.where(gt1, m1, jnp.maximum(m2, v))
                            n_i1 = jnp.where(gt1, col, i1)
                            n_m1 = jnp.maximum(m1, v)
                            st[i] = (n_m1, n_m2, n_i1, n_i2)
                    return tuple(st)

                init = tuple((neg, neg, zeros, zeros) for _ in range(I))
                res = lax.fori_loop(0, D // U, dbody, init)
                for i in range(I):
                    m1, m2, i1, i2 = res[i]
                    r = jnp.exp(m2 - m1)
                    pred = (i1.astype(jnp.float32)
                            + i2.astype(jnp.float32) * r) / (1.0 + r)
                    out_v[pl.ds(c * G + (g0 + i) * L, L)] = pred

        start(0, buf_a, sem_a)

        def chunk_body(i, _):
            c0 = i * 2
            start(c0 + 1, buf_b, sem_b)
            wait(buf_a, sem_a)
            compute(buf_a, c0)

            @pl.when(i < NCH // 2 - 1)
            def _():
                start(c0 + 2, buf_a, sem_a)

            wait(buf_b, sem_b)
            compute(buf_b, c0 + 1)
            return 0

        lax.fori_loop(0, NCH // 2, chunk_body, 0)
        pltpu.sync_copy(out_v, out_hbm.at[pl.ds(row0, RPW)])

    return disp_kernel


_disp_kernel = _make_kernel()


@jax.jit
def kernel(cost):
    flat = _disp_kernel(cost.reshape(R_TOTAL * D))
    return flat.reshape(B, N)


# speedup vs baseline: 9.0733x; 1.7213x over previous
import functools

import jax
import jax.numpy as jnp
from jax import lax
from jax.experimental import pallas as pl
from jax.experimental.pallas import tpu as pltpu
from jax.experimental.pallas import tpu_sc as plsc

D = 192
B = 4
N = 65536
R_TOTAL = B * N
NC = 2
NS = 16
L = 16
NW = NC * NS
RPW = R_TOTAL // NW
G = 128
NCH = RPW // G
NGRP = G // L
I = 4
U = 4
P = D + 1


def _make_kernel():
    mesh = plsc.VectorSubcoreMesh(core_axis_name="c", subcore_axis_name="s")

    @functools.partial(
        pl.kernel,
        out_type=jax.ShapeDtypeStruct((R_TOTAL,), jnp.float32),
        mesh=mesh,
        compiler_params=pltpu.CompilerParams(
            needs_layout_passes=False, use_tc_tiling_on_sc=False),
        scratch_types=[
            pltpu.VMEM((G, P), jnp.float32),
            pltpu.VMEM((G, P), jnp.float32),
            pltpu.VMEM((RPW,), jnp.float32),
            pltpu.SemaphoreType.DMA,
            pltpu.SemaphoreType.DMA,
        ],
    )
    def disp_kernel(cost_hbm, out_hbm, buf_a, buf_b, out_v, sem_a, sem_b):
        wid = lax.axis_index("s") * NC + lax.axis_index("c")
        row0 = wid * RPW
        iota = lax.iota(jnp.int32, L)

        def start(c, buf, sem):
            pltpu.async_copy(cost_hbm.at[pl.ds(row0 + c * G, G)],
                             buf.at[:, pl.ds(0, D)], sem)

        def wait(buf, sem):
            pltpu.make_async_copy(cost_hbm.at[pl.ds(0, G)],
                                  buf.at[:, pl.ds(0, D)], sem).wait()

        neg = jnp.full((L,), -jnp.inf, dtype=jnp.float32)
        zeros = jnp.zeros((L,), dtype=jnp.int32)

        def compute(buf, c):
            for g0 in range(0, NGRP, I):
                rows = [iota + (g0 + i) * L for i in range(I)]

                def dbody(dd, carry):
                    st = list(carry)
                    for u in range(U):
                        d = dd * U + u
                        col = zeros + d
                        for i in range(I):
                            m1, m2, i1, i2 = st[i]
                            v = plsc.load_gather(buf, [rows[i], col])
                            gt1 = v > m1
                            gt2 = v > m2
                            n_i2 = jnp.where(gt1, i1, jnp.where(gt2, col, i2))
                            n_m2 = jnp.where(gt1, m1, jnp.maximum(m2, v))
                            n_i1 = jnp.where(gt1, col, i1)
                            n_m1 = jnp.maximum(m1, v)
                            st[i] = (n_m1, n_m2, n_i1, n_i2)
                    return tuple(st)

                init = tuple((neg, neg, zeros, zeros) for _ in range(I))
                res = lax.fori_loop(0, D // U, dbody, init)
                for i in range(I):
                    m1, m2, i1, i2 = res[i]
                    r = jnp.exp(m2 - m1)
                    pred = (i1.astype(jnp.float32)
                            + i2.astype(jnp.float32) * r) / (1.0 + r)
                    out_v[pl.ds(c * G + (g0 + i) * L, L)] = pred

        start(0, buf_a, sem_a)

        def chunk_body(i, _):
            c0 = i * 2
            start(c0 + 1, buf_b, sem_b)
            wait(buf_a, sem_a)
            compute(buf_a, c0)

            @pl.when(i < NCH // 2 - 1)
            def _():
                start(c0 + 2, buf_a, sem_a)

            wait(buf_b, sem_b)
            compute(buf_b, c0 + 1)
            return 0

        lax.fori_loop(0, NCH // 2, chunk_body, 0)
        pltpu.sync_copy(out_v, out_hbm.at[pl.ds(row0, RPW)])

    return disp_kernel


_disp_kernel = _make_kernel()


@jax.jit
def kernel(cost):
    flat = _disp_kernel(cost.reshape(R_TOTAL, D))
    return flat.reshape(B, N)


# speedup vs baseline: 36.6915x; 4.0439x over previous
import functools

import jax
import jax.numpy as jnp
from jax import lax
from jax.experimental import pallas as pl
from jax.experimental.pallas import tpu as pltpu
from jax.experimental.pallas import tpu_sc as plsc

D = 192
B = 4
N = 65536
DT = D // 8
NPX = B * N
NC = 2
NS = 16
L = 16
NW = NC * NS
PPW = NPX // NW
PB = 128
NCH = PPW // PB
NGRP = PB // L
I = 4


def _make_kernel():
    mesh = plsc.VectorSubcoreMesh(core_axis_name="c", subcore_axis_name="s")

    @functools.partial(
        pl.kernel,
        out_type=jax.ShapeDtypeStruct((NPX,), jnp.float32),
        mesh=mesh,
        compiler_params=pltpu.CompilerParams(needs_layout_passes=False),
        scratch_types=[
            pltpu.VMEM((DT, 8, PB), jnp.float32),
            pltpu.VMEM((DT, 8, PB), jnp.float32),
            pltpu.VMEM((PPW,), jnp.float32),
            pltpu.SemaphoreType.DMA,
            pltpu.SemaphoreType.DMA,
        ],
    )
    def disp_kernel(cost_hbm, out_hbm, buf_a, buf_b, out_v, sem_a, sem_b):
        wid = lax.axis_index("s") * NC + lax.axis_index("c")
        b = wid // 8
        nt0 = (wid % 8) * NCH

        def start(k, buf, sem):
            pltpu.async_copy(
                cost_hbm.at[b, :, :, pl.ds((nt0 + k) * PB, PB)], buf, sem)

        def wait(buf, sem):
            pltpu.make_async_copy(
                cost_hbm.at[0, :, :, pl.ds(0, PB)], buf, sem).wait()

        neg = jnp.full((L,), -jnp.inf, dtype=jnp.float32)
        zeros = jnp.zeros((L,), dtype=jnp.int32)
        ones = jnp.ones((L,), dtype=jnp.int32)

        def scan_top2(buf, k):
            for g0 in range(0, NGRP, I):

                def dtbody(dt, carry):
                    dvec, st = carry
                    st = list(st)
                    for ds in range(8):
                        for i in range(I):
                            m1, m2, i1, i2 = st[i]
                            v = buf[dt, ds, pl.ds((g0 + i) * L, L)]
                            gt1 = v > m1
                            gt2 = v > m2
                            n_i2 = jnp.where(gt1, i1, jnp.where(gt2, dvec, i2))
                            n_m2 = jnp.where(gt1, m1, jnp.maximum(m2, v))
                            n_i1 = jnp.where(gt1, dvec, i1)
                            n_m1 = jnp.maximum(m1, v)
                            st[i] = (n_m1, n_m2, n_i1, n_i2)
                        dvec = dvec + ones
                    return dvec, tuple(st)

                init = (zeros, tuple((neg, neg, zeros, zeros)
                                     for _ in range(I)))
                _, res = lax.fori_loop(0, DT, dtbody, init)
                for i in range(I):
                    m1, m2, i1, i2 = res[i]
                    r = jnp.exp(m2 - m1)
                    pred = (i1.astype(jnp.float32)
                            + i2.astype(jnp.float32) * r) / (1.0 + r)
                    out_v[pl.ds(k * PB + (g0 + i) * L, L)] = pred

        start(0, buf_a, sem_a)

        def chunk_body(i, _):
            c0 = i * 2
            start(c0 + 1, buf_b, sem_b)
            wait(buf_a, sem_a)
            scan_top2(buf_a, c0)

            @pl.when(i < NCH // 2 - 1)
            def _():
                start(c0 + 2, buf_a, sem_a)

            wait(buf_b, sem_b)
            scan_top2(buf_b, c0 + 1)
            return 0

        lax.fori_loop(0, NCH // 2, chunk_body, 0)
        pltpu.sync_copy(out_v, out_hbm.at[pl.ds(wid * PPW, PPW)])

    return disp_kernel


_disp_kernel = _make_kernel()


@jax.jit
def kernel(cost):
    cost_t = cost.transpose(0, 2, 1).reshape(B, DT, 8, N)
    flat = _disp_kernel(cost_t)
    return flat.reshape(B, N)


# speedup vs baseline: 36.7167x; 1.0007x over previous
import functools

import jax
import jax.numpy as jnp
from jax import lax
from jax.experimental import pallas as pl
from jax.experimental.pallas import tpu as pltpu
from jax.experimental.pallas import tpu_sc as plsc

D = 192
B = 4
N = 65536
DT = D // 8
NPX = B * N
NC = 2
NS = 16
L = 16
NW = NC * NS
PPW = NPX // NW
PB = 128
NCH = PPW // PB
NGRP = PB // L
I = 8


def _make_kernel():
    mesh = plsc.VectorSubcoreMesh(core_axis_name="c", subcore_axis_name="s")

    @functools.partial(
        pl.kernel,
        out_type=jax.ShapeDtypeStruct((NPX,), jnp.float32),
        mesh=mesh,
        compiler_params=pltpu.CompilerParams(needs_layout_passes=False),
        scratch_types=[
            pltpu.VMEM((DT, 8, PB), jnp.float32),
            pltpu.VMEM((DT, 8, PB), jnp.float32),
            pltpu.VMEM((PPW,), jnp.float32),
            pltpu.SemaphoreType.DMA,
            pltpu.SemaphoreType.DMA,
        ],
    )
    def disp_kernel(cost_hbm, out_hbm, buf_a, buf_b, out_v, sem_a, sem_b):
        wid = lax.axis_index("s") * NC + lax.axis_index("c")
        b = wid // 8
        nt0 = (wid % 8) * NCH

        def start(k, buf, sem):
            pltpu.async_copy(
                cost_hbm.at[b, :, :, pl.ds((nt0 + k) * PB, PB)], buf, sem)

        def wait(buf, sem):
            pltpu.make_async_copy(
                cost_hbm.at[0, :, :, pl.ds(0, PB)], buf, sem).wait()

        neg = jnp.full((L,), -jnp.inf, dtype=jnp.float32)
        zeros = jnp.zeros((L,), dtype=jnp.int32)
        ones = jnp.ones((L,), dtype=jnp.int32)

        def scan_top2(buf, k):
            for g0 in range(0, NGRP, I):

                def dtbody(dt, carry):
                    dvec, st = carry
                    st = list(st)
                    for ds in range(8):
                        for i in range(I):
                            m1, m2, i1, i2 = st[i]
                            v = buf[dt, ds, pl.ds((g0 + i) * L, L)]
                            gt1 = v > m1
                            gt2 = v > m2
                            n_i2 = jnp.where(gt1, i1, jnp.where(gt2, dvec, i2))
                            n_m2 = jnp.where(gt1, m1, jnp.maximum(m2, v))
                            n_i1 = jnp.where(gt1, dvec, i1)
                            n_m1 = jnp.maximum(m1, v)
                            st[i] = (n_m1, n_m2, n_i1, n_i2)
                        dvec = dvec + ones
                    return dvec, tuple(st)

                init = (zeros, tuple((neg, neg, zeros, zeros)
                                     for _ in range(I)))
                _, res = lax.fori_loop(0, DT, dtbody, init)
                for i in range(I):
                    m1, m2, i1, i2 = res[i]
                    r = jnp.exp(m2 - m1)
                    pred = (i1.astype(jnp.float32)
                            + i2.astype(jnp.float32) * r) / (1.0 + r)
                    out_v[pl.ds(k * PB + (g0 + i) * L, L)] = pred

        start(0, buf_a, sem_a)

        def chunk_body(i, _):
            c0 = i * 2
            start(c0 + 1, buf_b, sem_b)
            wait(buf_a, sem_a)
            scan_top2(buf_a, c0)

            @pl.when(i < NCH // 2 - 1)
            def _():
                start(c0 + 2, buf_a, sem_a)

            wait(buf_b, sem_b)
            scan_top2(buf_b, c0 + 1)
            return 0

        lax.fori_loop(0, NCH // 2, chunk_body, 0)
        pltpu.sync_copy(out_v, out_hbm.at[pl.ds(wid * PPW, PPW)])

    return disp_kernel


_disp_kernel = _make_kernel()


@jax.jit
def kernel(cost):
    cost_t = cost.transpose(0, 2, 1).reshape(B, DT, 8, N)
    flat = _disp_kernel(cost_t)
    return flat.reshape(B, N)


# speedup vs baseline: 47.0810x; 1.2823x over previous
import functools

import jax
import jax.numpy as jnp
from jax import lax
from jax.experimental import pallas as pl
from jax.experimental.pallas import tpu as pltpu
from jax.experimental.pallas import tpu_sc as plsc

D = 192
B = 4
N = 65536
DT = D // 8
NC = 2
NS = 16
L = 16
NW = NC * NS
PB = 128

M_SC = 13
NSN = 2048 * M_SC
WN = N - NSN
NCH = 2 * M_SC
PPW = NCH * PB
NGRP = PB // L
I = 8
BLKN = 1024


def _make_sc_kernel():
    mesh = plsc.VectorSubcoreMesh(core_axis_name="c", subcore_axis_name="s")

    @functools.partial(
        pl.kernel,
        out_type=jax.ShapeDtypeStruct((B * NSN,), jnp.float32),
        mesh=mesh,
        compiler_params=pltpu.CompilerParams(needs_layout_passes=False),
        scratch_types=[
            pltpu.VMEM((DT, 8, PB), jnp.float32),
            pltpu.VMEM((DT, 8, PB), jnp.float32),
            pltpu.VMEM((PPW,), jnp.float32),
            pltpu.SemaphoreType.DMA,
            pltpu.SemaphoreType.DMA,
        ],
    )
    def disp_kernel(cost_hbm, out_hbm, buf_a, buf_b, out_v, sem_a, sem_b):
        wid = lax.axis_index("s") * NC + lax.axis_index("c")
        b = wid // 8
        nt0 = (wid % 8) * NCH

        def start(k, buf, sem):
            pltpu.async_copy(
                cost_hbm.at[b, :, :, pl.ds((nt0 + k) * PB, PB)], buf, sem)

        def wait(buf, sem):
            pltpu.make_async_copy(
                cost_hbm.at[0, :, :, pl.ds(0, PB)], buf, sem).wait()

        neg = jnp.full((L,), -jnp.inf, dtype=jnp.float32)
        zeros = jnp.zeros((L,), dtype=jnp.int32)
        ones = jnp.ones((L,), dtype=jnp.int32)

        def scan_top2(buf, k):
            for g0 in range(0, NGRP, I):

                def dtbody(dt, carry):
                    dvec, st = carry
                    st = list(st)
                    for ds in range(8):
                        for i in range(I):
                            m1, m2, i1, i2 = st[i]
                            v = buf[dt, ds, pl.ds((g0 + i) * L, L)]
                            gt1 = v > m1
                            gt2 = v > m2
                            n_i2 = jnp.where(gt1, i1, jnp.where(gt2, dvec, i2))
                            n_m2 = jnp.where(gt1, m1, jnp.maximum(m2, v))
                            n_i1 = jnp.where(gt1, dvec, i1)
                            n_m1 = jnp.maximum(m1, v)
                            st[i] = (n_m1, n_m2, n_i1, n_i2)
                        dvec = dvec + ones
                    return dvec, tuple(st)

                init = (zeros, tuple((neg, neg, zeros, zeros)
                                     for _ in range(I)))
                _, res = lax.fori_loop(0, DT, dtbody, init)
                for i in range(I):
                    m1, m2, i1, i2 = res[i]
                    r = jnp.exp(m2 - m1)
                    pred = (i1.astype(jnp.float32)
                            + i2.astype(jnp.float32) * r) / (1.0 + r)
                    out_v[pl.ds(k * PB + (g0 + i) * L, L)] = pred

        start(0, buf_a, sem_a)

        def chunk_body(i, _):
            c0 = i * 2
            start(c0 + 1, buf_b, sem_b)
            wait(buf_a, sem_a)
            scan_top2(buf_a, c0)

            @pl.when(i < NCH // 2 - 1)
            def _():
                start(c0 + 2, buf_a, sem_a)

            wait(buf_b, sem_b)
            scan_top2(buf_b, c0 + 1)
            return 0

        lax.fori_loop(0, NCH // 2, chunk_body, 0)
        pltpu.sync_copy(out_v, out_hbm.at[pl.ds(wid * PPW, PPW)])

    return disp_kernel


def _tc_block(x_ref, o_ref):
    x = x_ref[0].reshape(D, BLKN)
    iota = lax.broadcasted_iota(jnp.int32, (D, BLKN), 0)
    big = jnp.int32(D)
    m1 = jnp.max(x, axis=0)
    i1 = jnp.min(jnp.where(x == m1[None, :], iota, big), axis=0)
    x2 = jnp.where(iota == i1[None, :], -jnp.inf, x)
    m2 = jnp.max(x2, axis=0)
    i2 = jnp.min(jnp.where(x2 == m2[None, :], iota, big), axis=0)
    r = jnp.exp(m2 - m1)
    o_ref[...] = (i1.astype(jnp.float32)
                  + i2.astype(jnp.float32) * r) / (1.0 + r)


_tc_call = pl.pallas_call(
    _tc_block,
    out_shape=jax.ShapeDtypeStruct((B * WN,), jnp.float32),
    grid=(B, WN // BLKN),
    in_specs=[pl.BlockSpec((1, DT, 8, BLKN),
                           lambda b, j: (b, 0, 0, NSN // BLKN + j))],
    out_specs=pl.BlockSpec((BLKN,), lambda b, j: (b * (WN // BLKN) + j)),
)

_disp_kernel = _make_sc_kernel()


@jax.jit
def kernel(cost):
    cost_t = cost.transpose(0, 2, 1).reshape(B, DT, 8, N)
    sc_out = _disp_kernel(cost_t).reshape(B, NSN)
    tc_out = _tc_call(cost_t).reshape(B, WN)
    return jnp.concatenate([sc_out, tc_out], axis=1)


# speedup vs baseline: 58.0315x; 1.2326x over previous
import functools

import jax
import jax.numpy as jnp
from jax import lax
from jax.experimental import pallas as pl
from jax.experimental.pallas import tpu as pltpu
from jax.experimental.pallas import tpu_sc as plsc

D = 192
B = 4
N = 65536
DT = D // 8
NC = 2
NS = 16
L = 16
NW = NC * NS
PB = 128

M_SC = 18
NSN = 2048 * M_SC
WN = N - NSN
NCH = 2 * M_SC
PPW = NCH * PB
NGRP = PB // L
I = 8
BLKN = 4096


def _make_sc_kernel():
    mesh = plsc.VectorSubcoreMesh(core_axis_name="c", subcore_axis_name="s")

    @functools.partial(
        pl.kernel,
        out_type=jax.ShapeDtypeStruct((B * NSN,), jnp.float32),
        mesh=mesh,
        compiler_params=pltpu.CompilerParams(needs_layout_passes=False),
        scratch_types=[
            pltpu.VMEM((DT, 8, PB), jnp.float32),
            pltpu.VMEM((DT, 8, PB), jnp.float32),
            pltpu.VMEM((PPW,), jnp.float32),
            pltpu.SemaphoreType.DMA,
            pltpu.SemaphoreType.DMA,
        ],
    )
    def disp_kernel(cost_hbm, out_hbm, buf_a, buf_b, out_v, sem_a, sem_b):
        wid = lax.axis_index("s") * NC + lax.axis_index("c")
        b = wid // 8
        nt0 = (wid % 8) * NCH

        def start(k, buf, sem):
            pltpu.async_copy(
                cost_hbm.at[b, :, :, pl.ds((nt0 + k) * PB, PB)], buf, sem)

        def wait(buf, sem):
            pltpu.make_async_copy(
                cost_hbm.at[0, :, :, pl.ds(0, PB)], buf, sem).wait()

        neg = jnp.full((L,), -jnp.inf, dtype=jnp.float32)
        zeros = jnp.zeros((L,), dtype=jnp.int32)
        ones = jnp.ones((L,), dtype=jnp.int32)

        def scan_top2(buf, k):
            for g0 in range(0, NGRP, I):

                def dtbody(dt, carry):
                    dvec, st = carry
                    st = list(st)
                    for ds in range(8):
                        for i in range(I):
                            m1, m2, i1, i2 = st[i]
                            v = buf[dt, ds, pl.ds((g0 + i) * L, L)]
                            gt1 = v > m1
                            gt2 = v > m2
                            n_i2 = jnp.where(gt1, i1, jnp.where(gt2, dvec, i2))
                            n_m2 = jnp.where(gt1, m1, jnp.maximum(m2, v))
                            n_i1 = jnp.where(gt1, dvec, i1)
                            n_m1 = jnp.maximum(m1, v)
                            st[i] = (n_m1, n_m2, n_i1, n_i2)
                        dvec = dvec + ones
                    return dvec, tuple(st)

                init = (zeros, tuple((neg, neg, zeros, zeros)
                                     for _ in range(I)))
                _, res = lax.fori_loop(0, DT, dtbody, init)
                for i in range(I):
                    m1, m2, i1, i2 = res[i]
                    r = jnp.exp(m2 - m1)
                    pred = (i1.astype(jnp.float32)
                            + i2.astype(jnp.float32) * r) / (1.0 + r)
                    out_v[pl.ds(k * PB + (g0 + i) * L, L)] = pred

        start(0, buf_a, sem_a)

        def chunk_body(i, _):
            c0 = i * 2
            start(c0 + 1, buf_b, sem_b)
            wait(buf_a, sem_a)
            scan_top2(buf_a, c0)

            @pl.when(i < NCH // 2 - 1)
            def _():
                start(c0 + 2, buf_a, sem_a)

            wait(buf_b, sem_b)
            scan_top2(buf_b, c0 + 1)
            return 0

        lax.fori_loop(0, NCH // 2, chunk_body, 0)
        pltpu.sync_copy(out_v, out_hbm.at[pl.ds(wid * PPW, PPW)])

    return disp_kernel


def _tc_block(x_ref, o_ref):
    x = x_ref[0].reshape(D, BLKN)
    iota = lax.broadcasted_iota(jnp.int32, (D, BLKN), 0).astype(jnp.float32)
    big = jnp.float32(D)
    m1 = jnp.max(x, axis=0)
    i1 = jnp.min(jnp.where(x == m1[None, :], iota, big), axis=0)
    x2 = jnp.where(iota == i1[None, :], -jnp.inf, x)
    m2 = jnp.max(x2, axis=0)
    i2 = jnp.min(jnp.where(x2 == m2[None, :], iota, big), axis=0)
    r = jnp.exp(m2 - m1)
    o_ref[...] = (i1 + i2 * r) / (1.0 + r)


_tc_call = pl.pallas_call(
    _tc_block,
    out_shape=jax.ShapeDtypeStruct((B * WN,), jnp.float32),
    grid=(B, WN // BLKN),
    in_specs=[pl.BlockSpec((1, DT, 8, BLKN),
                           lambda b, j: (b, 0, 0, NSN // BLKN + j))],
    out_specs=pl.BlockSpec((BLKN,), lambda b, j: (b * (WN // BLKN) + j)),
)

_disp_kernel = _make_sc_kernel()


@jax.jit
def kernel(cost):
    cost_t = cost.transpose(0, 2, 1).reshape(B, DT, 8, N)
    sc_out = _disp_kernel(cost_t).reshape(B, NSN)
    tc_out = _tc_call(cost_t).reshape(B, WN)
    return jnp.concatenate([sc_out, tc_out], axis=1)


# speedup vs baseline: 77.7010x; 1.3389x over previous
import functools

import jax
import jax.numpy as jnp
from jax import lax
from jax.experimental import pallas as pl
from jax.experimental.pallas import tpu as pltpu
from jax.experimental.pallas import tpu_sc as plsc

D = 192
B = 4
N = 65536
DT = D // 8
NC = 2
NS = 16
L = 16
NW = NC * NS
PB = 128

M_SC = 12
NSN = 2048 * M_SC
WN = N - NSN
NCH = 2 * M_SC
PPW = NCH * PB
NGRP = PB // L
I = 8
BLKN = 8192


def _make_sc_kernel():
    mesh = plsc.VectorSubcoreMesh(core_axis_name="c", subcore_axis_name="s")

    @functools.partial(
        pl.kernel,
        out_type=jax.ShapeDtypeStruct((B * NSN,), jnp.float32),
        mesh=mesh,
        compiler_params=pltpu.CompilerParams(needs_layout_passes=False),
        scratch_types=[
            pltpu.VMEM((DT, 8, PB), jnp.float32),
            pltpu.VMEM((DT, 8, PB), jnp.float32),
            pltpu.VMEM((PPW,), jnp.float32),
            pltpu.SemaphoreType.DMA,
            pltpu.SemaphoreType.DMA,
        ],
    )
    def disp_kernel(cost_hbm, out_hbm, buf_a, buf_b, out_v, sem_a, sem_b):
        wid = lax.axis_index("s") * NC + lax.axis_index("c")
        b = wid // 8
        nt0 = (wid % 8) * NCH

        def start(k, buf, sem):
            pltpu.async_copy(
                cost_hbm.at[b, :, :, pl.ds((nt0 + k) * PB, PB)], buf, sem)

        def wait(buf, sem):
            pltpu.make_async_copy(
                cost_hbm.at[0, :, :, pl.ds(0, PB)], buf, sem).wait()

        neg = jnp.full((L,), -jnp.inf, dtype=jnp.float32)
        zeros = jnp.zeros((L,), dtype=jnp.int32)
        ones = jnp.ones((L,), dtype=jnp.int32)

        def scan_top2(buf, k):
            for g0 in range(0, NGRP, I):

                def dtbody(dt, carry):
                    dvec, st = carry
                    st = list(st)
                    for ds in range(8):
                        for i in range(I):
                            m1, m2, i1, i2 = st[i]
                            v = buf[dt, ds, pl.ds((g0 + i) * L, L)]
                            gt1 = v > m1
                            gt2 = v > m2
                            n_i2 = jnp.where(gt1, i1, jnp.where(gt2, dvec, i2))
                            n_m2 = jnp.where(gt1, m1, jnp.maximum(m2, v))
                            n_i1 = jnp.where(gt1, dvec, i1)
                            n_m1 = jnp.maximum(m1, v)
                            st[i] = (n_m1, n_m2, n_i1, n_i2)
                        dvec = dvec + ones
                    return dvec, tuple(st)

                init = (zeros, tuple((neg, neg, zeros, zeros)
                                     for _ in range(I)))
                _, res = lax.fori_loop(0, DT, dtbody, init)
                for i in range(I):
                    m1, m2, i1, i2 = res[i]
                    r = jnp.exp(m2 - m1)
                    pred = (i1.astype(jnp.float32)
                            + i2.astype(jnp.float32) * r) / (1.0 + r)
                    out_v[pl.ds(k * PB + (g0 + i) * L, L)] = pred

        start(0, buf_a, sem_a)

        def chunk_body(i, _):
            c0 = i * 2
            start(c0 + 1, buf_b, sem_b)
            wait(buf_a, sem_a)
            scan_top2(buf_a, c0)

            @pl.when(i < NCH // 2 - 1)
            def _():
                start(c0 + 2, buf_a, sem_a)

            wait(buf_b, sem_b)
            scan_top2(buf_b, c0 + 1)
            return 0

        lax.fori_loop(0, NCH // 2, chunk_body, 0)
        pltpu.sync_copy(out_v, out_hbm.at[pl.ds(wid * PPW, PPW)])

    return disp_kernel


def _tc_block(x_ref, o_ref):
    x = x_ref[0].reshape(D, BLKN)
    iota = lax.broadcasted_iota(jnp.int32, (D, BLKN), 0).astype(jnp.float32)
    big = jnp.float32(D)
    m1 = jnp.max(x, axis=0)
    i1 = jnp.min(jnp.where(x == m1[None, :], iota, big), axis=0)
    x2 = jnp.where(iota == i1[None, :], -jnp.inf, x)
    m2 = jnp.max(x2, axis=0)
    i2 = jnp.min(jnp.where(x2 == m2[None, :], iota, big), axis=0)
    r = jnp.exp(m2 - m1)
    o_ref[...] = (i1 + i2 * r) / (1.0 + r)


_tc_call = pl.pallas_call(
    _tc_block,
    out_shape=jax.ShapeDtypeStruct((B * WN,), jnp.float32),
    grid=(B, WN // BLKN),
    in_specs=[pl.BlockSpec((1, DT, 8, BLKN),
                           lambda b, j: (b, 0, 0, NSN // BLKN + j))],
    out_specs=pl.BlockSpec((BLKN,), lambda b, j: (b * (WN // BLKN) + j)),
)

_disp_kernel = _make_sc_kernel()


@jax.jit
def kernel(cost):
    cost_t = cost.transpose(0, 2, 1).reshape(B, DT, 8, N)
    sc_out = _disp_kernel(cost_t).reshape(B, NSN)
    tc_out = _tc_call(cost_t).reshape(B, WN)
    return jnp.concatenate([sc_out, tc_out], axis=1)
